# baseline (speedup 1.0000x reference)
import numpy as np
import jax
import jax.numpy as jnp
from jax.experimental import pallas as pl
from jax.experimental.pallas import tpu as pltpu

_C, _T, _H, _W = 3, 32, 384, 384
_ALPHA = 4
_NSLOW = _T // _ALPHA
_IDX = tuple(int(v) for v in np.linspace(0.0, _T - 1, _NSLOW).astype(np.float32))
_OCT = 8
assert all(_IDX[2 * o] - _OCT * o == max(0, o - 1) for o in range(_T // _OCT))
assert all(_IDX[2 * o + 1] - _OCT * o == o + 4 for o in range(_T // _OCT))


def _body(in_ref, slow_ref, fast_ref, fast_sem):
    o = pl.program_id(0)
    cp = pltpu.make_async_copy(
        in_ref, fast_ref.at[:, pl.ds(o * _OCT, _OCT)], fast_sem)
    cp.start()
    off0 = jnp.maximum(0, o - 1)
    off1 = o + 4
    slow_ref[:, pl.ds(0, 1)] = in_ref[:, pl.ds(off0, 1)]
    slow_ref[:, pl.ds(1, 1)] = in_ref[:, pl.ds(off1, 1)]
    cp.wait()


def kernel(frames):
    slow, fast = pl.pallas_call(
        _body,
        grid=(_T // _OCT,),
        in_specs=[pl.BlockSpec((_C, _OCT, _H, _W), lambda o: (0, o, 0, 0))],
        out_specs=[
            pl.BlockSpec((_C, 2, _H, _W), lambda o: (0, o, 0, 0)),
            pl.BlockSpec(memory_space=pl.ANY),
        ],
        out_shape=[
            jax.ShapeDtypeStruct((_C, _NSLOW, _H, _W), frames.dtype),
            jax.ShapeDtypeStruct((_C, _T, _H, _W), frames.dtype),
        ],
        scratch_shapes=[pltpu.SemaphoreType.DMA],
        compiler_params=pltpu.CompilerParams(
            vmem_limit_bytes=100 * 1024 * 1024,
        ),
    )(frames)
    return (slow, fast)


# speedup vs baseline: 1.0769x; 1.0769x over previous
import numpy as np
import jax
import jax.numpy as jnp
from jax.experimental import pallas as pl
from jax.experimental.pallas import tpu as pltpu

_C, _T, _H, _W = 3, 32, 384, 384
_ALPHA = 4
_NSLOW = _T // _ALPHA
_IDX = tuple(int(v) for v in np.linspace(0.0, _T - 1, _NSLOW).astype(np.float32))
_OCT = 8
assert all(_IDX[2 * o] - _OCT * o == max(0, o - 1) for o in range(_T // _OCT))
assert all(_IDX[2 * o + 1] - _OCT * o == o + 4 for o in range(_T // _OCT))


def _body(in_ref, slow_ref, fast_ref):
    o = pl.program_id(0)
    x = in_ref[...]
    fast_ref[...] = x
    off0 = jnp.maximum(0, o - 1)
    off1 = o + 4
    slow_ref[:, pl.ds(0, 1)] = in_ref[:, pl.ds(off0, 1)]
    slow_ref[:, pl.ds(1, 1)] = in_ref[:, pl.ds(off1, 1)]


def kernel(frames):
    slow, fast = pl.pallas_call(
        _body,
        grid=(_T // _OCT,),
        in_specs=[pl.BlockSpec((_C, _OCT, _H, _W), lambda o: (0, o, 0, 0))],
        out_specs=[
            pl.BlockSpec((_C, 2, _H, _W), lambda o: (0, o, 0, 0)),
            pl.BlockSpec((_C, _OCT, _H, _W), lambda o: (0, o, 0, 0)),
        ],
        out_shape=[
            jax.ShapeDtypeStruct((_C, _NSLOW, _H, _W), frames.dtype),
            jax.ShapeDtypeStruct((_C, _T, _H, _W), frames.dtype),
        ],
        compiler_params=pltpu.CompilerParams(
            vmem_limit_bytes=100 * 1024 * 1024,
        ),
    )(frames)
    return (slow, fast)
